# baseline (speedup 1.0000x reference)
import functools

import jax
import jax.numpy as jnp
from jax import lax
from jax.experimental import pallas as pl
from jax.experimental.pallas import tpu as pltpu
from jax.experimental.pallas import tpu_sc as plsc

B = 16384
D = 64
H1 = 256
H2 = 128
VP = 50000

NC = 2
NS = 16
NW = NC * NS
IDX_CHUNK = 128

NCH = 2
BC = B // NCH
B_PER_W = BC // NW
N_CHUNKS = B_PER_W // IDX_CHUNK


RBLK = 1000


def _tc_repack_body(u_ref, i_ref, uo_ref, io_ref):
    for src, dst in ((u_ref, uo_ref), (i_ref, io_ref)):
        x = src[...].reshape(RBLK, 2, D)
        dst[:, pl.ds(0, D)] = x[:, 0, :]
        dst[:, pl.ds(D, D)] = x[:, 1, :]


def _tc_repack(utab, itab):
    grid = (VP // RBLK,)
    return pl.pallas_call(
        _tc_repack_body,
        grid=grid,
        in_specs=[
            pl.BlockSpec((2 * RBLK, D), lambda i: (i, 0)),
            pl.BlockSpec((2 * RBLK, D), lambda i: (i, 0)),
        ],
        out_specs=[
            pl.BlockSpec((RBLK, 2 * D), lambda i: (i, 0)),
            pl.BlockSpec((RBLK, 2 * D), lambda i: (i, 0)),
        ],
        out_shape=[
            jax.ShapeDtypeStruct((VP, 2 * D), jnp.float32),
            jax.ShapeDtypeStruct((VP, 2 * D), jnp.float32),
        ],
    )(utab, itab)


def _sc_gather_body(pu_hbm, pi_hbm, utab_hbm, itab_hbm, out_hbm,
                    uidx_v, iidx_v, urows_v, irows_v, usem, isem, osem):
    wid = lax.axis_index("s") * NC + lax.axis_index("c")
    base = wid * B_PER_W
    pltpu.sync_copy(pu_hbm.at[pl.ds(base, B_PER_W)], uidx_v)
    pltpu.sync_copy(pi_hbm.at[pl.ds(base, B_PER_W)], iidx_v)
    out_copies = [None, None]
    for j in range(N_CHUNKS):
        s = j & 1
        if out_copies[s] is not None:
            out_copies[s][0].wait()
            out_copies[s][1].wait()
        r = pl.ds(j * IDX_CHUNK, IDX_CHUNK)
        gu = pltpu.async_copy(utab_hbm.at[uidx_v.at[r]], urows_v.at[s], usem)
        gi = pltpu.async_copy(itab_hbm.at[iidx_v.at[r]], irows_v.at[s], isem)
        gu.wait()
        gi.wait()
        rows = pl.ds(base + j * IDX_CHUNK, IDX_CHUNK)
        cu = pltpu.async_copy(
            urows_v.at[s], out_hbm.at[rows, pl.ds(0, 2 * D)], osem)
        ci = pltpu.async_copy(
            irows_v.at[s], out_hbm.at[rows, pl.ds(2 * D, 2 * D)], osem)
        out_copies[s] = (cu, ci)
    for pair in out_copies:
        if pair is not None:
            pair[0].wait()
            pair[1].wait()


_sc_gather = functools.partial(
    pl.kernel,
    out_type=jax.ShapeDtypeStruct((BC, 4 * D), jnp.float32),
    mesh=plsc.VectorSubcoreMesh(core_axis_name="c", subcore_axis_name="s"),
    scratch_types=[
        pltpu.VMEM((B_PER_W,), jnp.int32),
        pltpu.VMEM((B_PER_W,), jnp.int32),
        pltpu.VMEM((2, IDX_CHUNK, 2 * D), jnp.float32),
        pltpu.VMEM((2, IDX_CHUNK, 2 * D), jnp.float32),
        pltpu.SemaphoreType.DMA,
        pltpu.SemaphoreType.DMA,
        pltpu.SemaphoreType.DMA,
    ],
    compiler_params=pltpu.CompilerParams(use_tc_tiling_on_sc=True),
)(_sc_gather_body)


BB = 2048


def _tc_mlp_body(pairs_ref, uid_ref, iid_ref, w1ui_ref, w1d_ref, b1_ref,
                 w2_ref, b2_ref, w3_ref, b3_ref, out_ref):
    pairs = pairs_ref[...]
    upar = (uid_ref[...] & 1)[:, None] == 1
    ipar = (iid_ref[...] & 1)[:, None] == 1
    u = jnp.where(upar, pairs[:, D:2 * D], pairs[:, :D])
    it = jnp.where(ipar, pairs[:, 3 * D:], pairs[:, 2 * D:3 * D])
    ui = jnp.concatenate([u, it], axis=1)
    dot = jnp.sum(u * it, axis=1, keepdims=True)
    h = lax.dot_general(ui, w1ui_ref[...], (((1,), (1,)), ((), ())),
                        preferred_element_type=jnp.float32)
    h += dot * w1d_ref[...][None, :] + b1_ref[...][None, :]
    h = jnp.maximum(h, 0.0)
    h2 = lax.dot_general(h, w2_ref[...], (((1,), (1,)), ((), ())),
                         preferred_element_type=jnp.float32)
    h2 = jnp.maximum(h2 + b2_ref[...][None, :], 0.0)
    p = jnp.sum(h2 * w3_ref[...][None, :], axis=1) + b3_ref[0]
    out_ref[...] = p


def _tc_mlp(pairs, uid, iid, w1ui, w1d, b1, w2, b2, w3, b3):
    n = pairs.shape[0]
    grid = (n // BB,)
    return pl.pallas_call(
        _tc_mlp_body,
        grid=grid,
        in_specs=[
            pl.BlockSpec((BB, 4 * D), lambda i: (i, 0)),
            pl.BlockSpec((BB,), lambda i: (i,)),
            pl.BlockSpec((BB,), lambda i: (i,)),
            pl.BlockSpec((H1, 2 * D), lambda i: (0, 0)),
            pl.BlockSpec((H1,), lambda i: (0,)),
            pl.BlockSpec((H1,), lambda i: (0,)),
            pl.BlockSpec((H2, H1), lambda i: (0, 0)),
            pl.BlockSpec((H2,), lambda i: (0,)),
            pl.BlockSpec((H2,), lambda i: (0,)),
            pl.BlockSpec(memory_space=pltpu.SMEM),
        ],
        out_specs=pl.BlockSpec((BB,), lambda i: (i,)),
        out_shape=jax.ShapeDtypeStruct((n,), jnp.float32),
    )(pairs, uid, iid, w1ui, w1d, b1, w2, b2, w3, b3)


def kernel(user_id, item_id, user_table, item_table, W1, b1, W2, b2, W3, b3):
    uid = user_id.astype(jnp.int32)
    iid = item_id.astype(jnp.int32)
    utab2, itab2 = _tc_repack(user_table, item_table)
    w1ui = W1[:, :2 * D]
    w1d = W1[:, 2 * D]
    w3 = W3[0]
    outs = []
    for c in range(NCH):
        sl = slice(c * BC, (c + 1) * BC)
        uc, ic = uid[sl], iid[sl]
        pairs = _sc_gather(uc >> 1, ic >> 1, utab2, itab2)
        outs.append(_tc_mlp(pairs, uc, ic, w1ui, w1d, b1, W2, b2, w3, b3))
    return jnp.concatenate(outs) if NCH > 1 else outs[0]


# speedup vs baseline: 1.2434x; 1.2434x over previous
import functools

import jax
import jax.numpy as jnp
from jax import lax
from jax.experimental import pallas as pl
from jax.experimental.pallas import tpu as pltpu
from jax.experimental.pallas import tpu_sc as plsc

B = 16384
D = 64
H1 = 256
H2 = 128
VP = 50000

NC = 2
NS = 16
NW = NC * NS
IDX_CHUNK = 128

NCH = 4
BC = B // NCH
B_PER_W = BC // NW
N_CHUNKS = B_PER_W // IDX_CHUNK


def _sc_gather_body(pu_hbm, pi_hbm, utab_hbm, itab_hbm, out_hbm,
                    uidx_v, iidx_v, urows_v, irows_v, usem, isem, osem):
    wid = lax.axis_index("s") * NC + lax.axis_index("c")
    base = wid * B_PER_W
    pltpu.sync_copy(pu_hbm.at[pl.ds(base, B_PER_W)], uidx_v)
    pltpu.sync_copy(pi_hbm.at[pl.ds(base, B_PER_W)], iidx_v)
    out_copies = [None, None]
    for j in range(N_CHUNKS):
        s = j & 1
        if out_copies[s] is not None:
            out_copies[s][0].wait()
            out_copies[s][1].wait()
        r = pl.ds(j * IDX_CHUNK, IDX_CHUNK)
        gu = pltpu.async_copy(utab_hbm.at[uidx_v.at[r]], urows_v.at[s], usem)
        gi = pltpu.async_copy(itab_hbm.at[iidx_v.at[r]], irows_v.at[s], isem)
        gu.wait()
        gi.wait()
        rows = pl.ds(base + j * IDX_CHUNK, IDX_CHUNK)
        cu = pltpu.async_copy(
            urows_v.at[s], out_hbm.at[rows, pl.ds(0, 2 * D)], osem)
        ci = pltpu.async_copy(
            irows_v.at[s], out_hbm.at[rows, pl.ds(2 * D, 2 * D)], osem)
        out_copies[s] = (cu, ci)
    for pair in out_copies:
        if pair is not None:
            pair[0].wait()
            pair[1].wait()


_sc_gather = functools.partial(
    pl.kernel,
    out_type=jax.ShapeDtypeStruct((BC, 4 * D), jnp.float32),
    mesh=plsc.VectorSubcoreMesh(core_axis_name="c", subcore_axis_name="s"),
    scratch_types=[
        pltpu.VMEM((B_PER_W,), jnp.int32),
        pltpu.VMEM((B_PER_W,), jnp.int32),
        pltpu.VMEM((2, IDX_CHUNK, 2 * D), jnp.float32),
        pltpu.VMEM((2, IDX_CHUNK, 2 * D), jnp.float32),
        pltpu.SemaphoreType.DMA,
        pltpu.SemaphoreType.DMA,
        pltpu.SemaphoreType.DMA,
    ],
    compiler_params=pltpu.CompilerParams(use_tc_tiling_on_sc=True),
)(_sc_gather_body)


BB = 2048


def _tc_mlp_body(pairs_ref, uid_ref, iid_ref, w1ui_ref, w1d_ref, b1_ref,
                 w2_ref, b2_ref, w3_ref, b3_ref, out_ref):
    pairs = pairs_ref[...]
    upar = (uid_ref[...] & 1)[:, None] == 1
    ipar = (iid_ref[...] & 1)[:, None] == 1
    u = jnp.where(upar, pairs[:, D:2 * D], pairs[:, :D])
    it = jnp.where(ipar, pairs[:, 3 * D:], pairs[:, 2 * D:3 * D])
    ui = jnp.concatenate([u, it], axis=1)
    dot = jnp.sum(u * it, axis=1, keepdims=True)
    h = lax.dot_general(ui, w1ui_ref[...], (((1,), (1,)), ((), ())),
                        preferred_element_type=jnp.float32)
    h += dot * w1d_ref[...][None, :] + b1_ref[...][None, :]
    h = jnp.maximum(h, 0.0)
    h2 = lax.dot_general(h, w2_ref[...], (((1,), (1,)), ((), ())),
                         preferred_element_type=jnp.float32)
    h2 = jnp.maximum(h2 + b2_ref[...][None, :], 0.0)
    p = jnp.sum(h2 * w3_ref[...][None, :], axis=1) + b3_ref[0]
    out_ref[...] = p


def _tc_mlp(pairs, uid, iid, w1ui, w1d, b1, w2, b2, w3, b3):
    n = pairs.shape[0]
    grid = (n // BB,)
    return pl.pallas_call(
        _tc_mlp_body,
        grid=grid,
        in_specs=[
            pl.BlockSpec((BB, 4 * D), lambda i: (i, 0)),
            pl.BlockSpec((BB,), lambda i: (i,)),
            pl.BlockSpec((BB,), lambda i: (i,)),
            pl.BlockSpec((H1, 2 * D), lambda i: (0, 0)),
            pl.BlockSpec((H1,), lambda i: (0,)),
            pl.BlockSpec((H1,), lambda i: (0,)),
            pl.BlockSpec((H2, H1), lambda i: (0, 0)),
            pl.BlockSpec((H2,), lambda i: (0,)),
            pl.BlockSpec((H2,), lambda i: (0,)),
            pl.BlockSpec(memory_space=pltpu.SMEM),
        ],
        out_specs=pl.BlockSpec((BB,), lambda i: (i,)),
        out_shape=jax.ShapeDtypeStruct((n,), jnp.float32),
    )(pairs, uid, iid, w1ui, w1d, b1, w2, b2, w3, b3)


def kernel(user_id, item_id, user_table, item_table, W1, b1, W2, b2, W3, b3):
    uid = user_id.astype(jnp.int32)
    iid = item_id.astype(jnp.int32)
    utab2, itab2 = lax.optimization_barrier(
        (user_table.reshape(VP, 2 * D), item_table.reshape(VP, 2 * D)))
    w1ui = W1[:, :2 * D]
    w1d = W1[:, 2 * D]
    w3 = W3[0]
    outs = []
    for c in range(NCH):
        sl = slice(c * BC, (c + 1) * BC)
        uc, ic = uid[sl], iid[sl]
        pairs = _sc_gather(uc >> 1, ic >> 1, utab2, itab2)
        outs.append(_tc_mlp(pairs, uc, ic, w1ui, w1d, b1, W2, b2, w3, b3))
    return jnp.concatenate(outs) if NCH > 1 else outs[0]


# speedup vs baseline: 1.2437x; 1.0002x over previous
import functools

import jax
import jax.numpy as jnp
from jax import lax
from jax.experimental import pallas as pl
from jax.experimental.pallas import tpu as pltpu
from jax.experimental.pallas import tpu_sc as plsc

B = 16384
D = 64
H1 = 256
H2 = 128
VP = 50000

NC = 2
NS = 16
NW = NC * NS
IDX_CHUNK = 128

NCH = 2
BC = B // NCH
B_PER_W = BC // NW
N_CHUNKS = B_PER_W // IDX_CHUNK


def _sc_gather_body(pu_hbm, pi_hbm, utab_hbm, itab_hbm, out_hbm,
                    uidx_v, iidx_v, urows_v, irows_v, usem, isem, osem):
    wid = lax.axis_index("s") * NC + lax.axis_index("c")
    base = wid * B_PER_W
    pltpu.sync_copy(pu_hbm.at[pl.ds(base, B_PER_W)], uidx_v)
    pltpu.sync_copy(pi_hbm.at[pl.ds(base, B_PER_W)], iidx_v)
    out_copies = [None, None]
    for j in range(N_CHUNKS):
        s = j & 1
        if out_copies[s] is not None:
            out_copies[s][0].wait()
            out_copies[s][1].wait()
        r = pl.ds(j * IDX_CHUNK, IDX_CHUNK)
        gu = pltpu.async_copy(utab_hbm.at[uidx_v.at[r]], urows_v.at[s], usem)
        gi = pltpu.async_copy(itab_hbm.at[iidx_v.at[r]], irows_v.at[s], isem)
        gu.wait()
        gi.wait()
        rows = pl.ds(base + j * IDX_CHUNK, IDX_CHUNK)
        cu = pltpu.async_copy(
            urows_v.at[s], out_hbm.at[rows, pl.ds(0, 2 * D)], osem)
        ci = pltpu.async_copy(
            irows_v.at[s], out_hbm.at[rows, pl.ds(2 * D, 2 * D)], osem)
        out_copies[s] = (cu, ci)
    for pair in out_copies:
        if pair is not None:
            pair[0].wait()
            pair[1].wait()


_sc_gather = functools.partial(
    pl.kernel,
    out_type=jax.ShapeDtypeStruct((BC, 4 * D), jnp.float32),
    mesh=plsc.VectorSubcoreMesh(core_axis_name="c", subcore_axis_name="s"),
    scratch_types=[
        pltpu.VMEM((B_PER_W,), jnp.int32),
        pltpu.VMEM((B_PER_W,), jnp.int32),
        pltpu.VMEM((2, IDX_CHUNK, 2 * D), jnp.float32),
        pltpu.VMEM((2, IDX_CHUNK, 2 * D), jnp.float32),
        pltpu.SemaphoreType.DMA,
        pltpu.SemaphoreType.DMA,
        pltpu.SemaphoreType.DMA,
    ],
    compiler_params=pltpu.CompilerParams(use_tc_tiling_on_sc=True),
)(_sc_gather_body)


BB = 2048


def _tc_mlp_body(pairs_ref, uid_ref, iid_ref, w1ui_ref, w1d_ref, b1_ref,
                 w2_ref, b2_ref, w3_ref, b3_ref, out_ref):
    pairs = pairs_ref[...]
    upar = (uid_ref[...] & 1)[:, None] == 1
    ipar = (iid_ref[...] & 1)[:, None] == 1
    u = jnp.where(upar, pairs[:, D:2 * D], pairs[:, :D])
    it = jnp.where(ipar, pairs[:, 3 * D:], pairs[:, 2 * D:3 * D])
    ui = jnp.concatenate([u, it], axis=1)
    dot = jnp.sum(u * it, axis=1, keepdims=True)
    h = lax.dot_general(ui, w1ui_ref[...], (((1,), (1,)), ((), ())),
                        preferred_element_type=jnp.float32)
    h += dot * w1d_ref[...][None, :] + b1_ref[...][None, :]
    h = jnp.maximum(h, 0.0)
    h2 = lax.dot_general(h, w2_ref[...], (((1,), (1,)), ((), ())),
                         preferred_element_type=jnp.float32)
    h2 = jnp.maximum(h2 + b2_ref[...][None, :], 0.0)
    p = jnp.sum(h2 * w3_ref[...][None, :], axis=1) + b3_ref[0]
    out_ref[...] = p


def _tc_mlp(pairs, uid, iid, w1ui, w1d, b1, w2, b2, w3, b3):
    n = pairs.shape[0]
    grid = (n // BB,)
    return pl.pallas_call(
        _tc_mlp_body,
        grid=grid,
        in_specs=[
            pl.BlockSpec((BB, 4 * D), lambda i: (i, 0)),
            pl.BlockSpec((BB,), lambda i: (i,)),
            pl.BlockSpec((BB,), lambda i: (i,)),
            pl.BlockSpec((H1, 2 * D), lambda i: (0, 0)),
            pl.BlockSpec((H1,), lambda i: (0,)),
            pl.BlockSpec((H1,), lambda i: (0,)),
            pl.BlockSpec((H2, H1), lambda i: (0, 0)),
            pl.BlockSpec((H2,), lambda i: (0,)),
            pl.BlockSpec((H2,), lambda i: (0,)),
            pl.BlockSpec(memory_space=pltpu.SMEM),
        ],
        out_specs=pl.BlockSpec((BB,), lambda i: (i,)),
        out_shape=jax.ShapeDtypeStruct((n,), jnp.float32),
    )(pairs, uid, iid, w1ui, w1d, b1, w2, b2, w3, b3)


def kernel(user_id, item_id, user_table, item_table, W1, b1, W2, b2, W3, b3):
    uid = user_id.astype(jnp.int32)
    iid = item_id.astype(jnp.int32)
    utab2, itab2 = lax.optimization_barrier(
        (user_table.reshape(VP, 2 * D), item_table.reshape(VP, 2 * D)))
    w1ui = W1[:, :2 * D]
    w1d = W1[:, 2 * D]
    w3 = W3[0]
    outs = []
    for c in range(NCH):
        sl = slice(c * BC, (c + 1) * BC)
        uc, ic = uid[sl], iid[sl]
        pairs = _sc_gather(uc >> 1, ic >> 1, utab2, itab2)
        outs.append(_tc_mlp(pairs, uc, ic, w1ui, w1d, b1, W2, b2, w3, b3))
    return jnp.concatenate(outs) if NCH > 1 else outs[0]


# speedup vs baseline: 1.3468x; 1.0830x over previous
import functools

import jax
import jax.numpy as jnp
from jax import lax
from jax.experimental import pallas as pl
from jax.experimental.pallas import tpu as pltpu
from jax.experimental.pallas import tpu_sc as plsc

B = 16384
D = 64
H1 = 256
H2 = 128
VP = 50000

NC = 2
NS = 16
NW = NC * NS
IDX_CHUNK = 128

NCH = 2
BC = B // NCH
B_PER_W = BC // NW
N_CHUNKS = B_PER_W // IDX_CHUNK


def _sc_gather_body(pu_hbm, pi_hbm, utab_hbm, itab_hbm, out_hbm,
                    uidx_v, iidx_v, urows_v, irows_v, usem, isem, osem):
    wid = lax.axis_index("s") * NC + lax.axis_index("c")
    base = wid * B_PER_W
    pltpu.sync_copy(pu_hbm.at[pl.ds(base, B_PER_W)], uidx_v)
    pltpu.sync_copy(pi_hbm.at[pl.ds(base, B_PER_W)], iidx_v)
    out_copies = [None, None]
    for j in range(N_CHUNKS):
        s = j & 1
        if out_copies[s] is not None:
            out_copies[s][0].wait()
            out_copies[s][1].wait()
        r = pl.ds(j * IDX_CHUNK, IDX_CHUNK)
        gu = pltpu.async_copy(utab_hbm.at[uidx_v.at[r]], urows_v.at[s], usem)
        gi = pltpu.async_copy(itab_hbm.at[iidx_v.at[r]], irows_v.at[s], isem)
        gu.wait()
        gi.wait()
        rows = pl.ds(base + j * IDX_CHUNK, IDX_CHUNK)
        cu = pltpu.async_copy(
            urows_v.at[s], out_hbm.at[rows, pl.ds(0, 2 * D)], osem)
        ci = pltpu.async_copy(
            irows_v.at[s], out_hbm.at[rows, pl.ds(2 * D, 2 * D)], osem)
        out_copies[s] = (cu, ci)
    for pair in out_copies:
        if pair is not None:
            pair[0].wait()
            pair[1].wait()


_sc_gather = functools.partial(
    pl.kernel,
    out_type=jax.ShapeDtypeStruct((BC, 4 * D), jnp.float32),
    mesh=plsc.VectorSubcoreMesh(core_axis_name="c", subcore_axis_name="s"),
    scratch_types=[
        pltpu.VMEM((B_PER_W,), jnp.int32),
        pltpu.VMEM((B_PER_W,), jnp.int32),
        pltpu.VMEM((2, IDX_CHUNK, 2 * D), jnp.float32),
        pltpu.VMEM((2, IDX_CHUNK, 2 * D), jnp.float32),
        pltpu.SemaphoreType.DMA,
        pltpu.SemaphoreType.DMA,
        pltpu.SemaphoreType.DMA,
    ],
    compiler_params=pltpu.CompilerParams(use_tc_tiling_on_sc=True),
)(_sc_gather_body)


BB = 2048


def _tc_mlp_body(pairs_ref, w1ui_ref, w1d_ref, b1_ref,
                 w2_ref, b2_ref, w3_ref, b3_ref, out_ref):
    pairs = pairs_ref[...]
    u = pairs[:, :D]
    it = pairs[:, 2 * D:3 * D]
    ui = jnp.concatenate([u, it], axis=1)
    dot = jnp.sum(u * it, axis=1, keepdims=True)
    h = lax.dot_general(ui, w1ui_ref[...], (((1,), (1,)), ((), ())),
                        preferred_element_type=jnp.float32)
    h += dot * w1d_ref[...][None, :] + b1_ref[...][None, :]
    h = jnp.maximum(h, 0.0)
    h2 = lax.dot_general(h, w2_ref[...], (((1,), (1,)), ((), ())),
                         preferred_element_type=jnp.float32)
    h2 = jnp.maximum(h2 + b2_ref[...][None, :], 0.0)
    p = jnp.sum(h2 * w3_ref[...][None, :], axis=1) + b3_ref[0]
    out_ref[...] = p


def _tc_mlp(pairs, w1ui, w1d, b1, w2, b2, w3, b3):
    n = pairs.shape[0]
    grid = (n // BB,)
    return pl.pallas_call(
        _tc_mlp_body,
        grid=grid,
        in_specs=[
            pl.BlockSpec((BB, 4 * D), lambda i: (i, 0)),
            pl.BlockSpec((H1, 2 * D), lambda i: (0, 0)),
            pl.BlockSpec((H1,), lambda i: (0,)),
            pl.BlockSpec((H1,), lambda i: (0,)),
            pl.BlockSpec((H2, H1), lambda i: (0, 0)),
            pl.BlockSpec((H2,), lambda i: (0,)),
            pl.BlockSpec((H2,), lambda i: (0,)),
            pl.BlockSpec(memory_space=pltpu.SMEM),
        ],
        out_specs=pl.BlockSpec((BB,), lambda i: (i,)),
        out_shape=jax.ShapeDtypeStruct((n,), jnp.float32),
    )(pairs, w1ui, w1d, b1, w2, b2, w3, b3)


def kernel(user_id, item_id, user_table, item_table, W1, b1, W2, b2, W3, b3):
    uid = user_id.astype(jnp.int32)
    iid = item_id.astype(jnp.int32)
    utab2 = jnp.pad(user_table, ((0, 0), (0, D)))
    itab2 = jnp.pad(item_table, ((0, 0), (0, D)))
    w1ui = W1[:, :2 * D]
    w1d = W1[:, 2 * D]
    w3 = W3[0]
    outs = []
    for c in range(NCH):
        sl = slice(c * BC, (c + 1) * BC)
        uc, ic = uid[sl], iid[sl]
        pairs = _sc_gather(uc, ic, utab2, itab2)
        outs.append(_tc_mlp(pairs, w1ui, w1d, b1, W2, b2, w3, b3))
    return jnp.concatenate(outs) if NCH > 1 else outs[0]
